# baseline (speedup 1.0000x reference)
import jax
import jax.numpy as jnp
from jax.experimental import pallas as pl
from jax.experimental.pallas import tpu as pltpu

N = 4096
IN_F = 64
OUT_F = 64
BLOCK_ROWS = 512


def _gcn_kernel(inp_ref, w_ref, b_ref, adj_hbm, out_hbm, s_ref):
    s_ref[...] = jnp.dot(
        inp_ref[...], w_ref[...], preferred_element_type=jnp.float32
    ).astype(jnp.bfloat16)

    def inner(adj_blk_ref, out_blk_ref):
        t = jnp.dot(
            adj_blk_ref[...].astype(jnp.bfloat16),
            s_ref[...],
            preferred_element_type=jnp.float32,
        )
        out_blk_ref[...] = t + b_ref[...]

    pltpu.emit_pipeline(
        inner,
        grid=(N // BLOCK_ROWS,),
        in_specs=[pl.BlockSpec((BLOCK_ROWS, N), lambda i: (i, 0))],
        out_specs=[pl.BlockSpec((BLOCK_ROWS, OUT_F), lambda i: (i, 0))],
    )(adj_hbm, out_hbm)


def kernel(input, adj, W, b):
    b2 = b.reshape(1, OUT_F)
    return pl.pallas_call(
        _gcn_kernel,
        in_specs=[
            pl.BlockSpec(memory_space=pltpu.MemorySpace.VMEM),
            pl.BlockSpec(memory_space=pltpu.MemorySpace.VMEM),
            pl.BlockSpec(memory_space=pltpu.MemorySpace.VMEM),
            pl.BlockSpec(memory_space=pltpu.MemorySpace.HBM),
        ],
        out_specs=pl.BlockSpec(memory_space=pltpu.MemorySpace.HBM),
        out_shape=jax.ShapeDtypeStruct((N, OUT_F), jnp.float32),
        scratch_shapes=[
            pltpu.VMEM((N, OUT_F), jnp.bfloat16),
        ],
    )(input, W, b2, adj)
